# baseline (speedup 1.0000x reference)
import functools
import jax
import jax.numpy as jnp
from jax import lax
from jax.experimental import pallas as pl
from jax.experimental.pallas import tpu as pltpu
from jax.experimental.pallas import tpu_sc as plsc

H = 128
OBS_IN = 16
HEADS = 5
NN = 10000
NE = 160000
BATCH = 2
SEQ = 10
ACTION = 8

NC = 2
NS = 16
LANES = 16
KCH = 64
EP = 172032
EPT = EP // NS
NCHUNK = EPT // KCH
NSEG = 10240
TRASH = 10008
TROWS = BATCH * NN
ZROWS = NSEG // NS
SUP = 256
CPS = SUP // KCH
NSUP = EPT // SUP


def _dense_body(x_ref, w_ref, b_ref, o_ref, *, act):
    y = jnp.dot(x_ref[...], w_ref[...], preferred_element_type=jnp.float32)
    y = y + b_ref[...]
    if act:
        y = jnp.maximum(y, 0.0)
    o_ref[...] = y


def _dense(x, w, b, act=True):
    m, _ = x.shape
    n = w.shape[1]
    return pl.pallas_call(
        functools.partial(_dense_body, act=act),
        out_shape=jax.ShapeDtypeStruct((m, n), jnp.float32),
    )(x, w, b.reshape(1, n))


def _gru_body(x_ref, wn_ref, bn_ref, wih_ref, whh_ref, bih_ref, bhh_ref,
              o_ref):
    rows = x_ref.shape[0]
    h = jnp.zeros((rows, H), jnp.float32)
    for t in range(SEQ):
        xt = x_ref[:, t * OBS_IN:(t + 1) * OBS_IN]
        emb = jnp.dot(xt, wn_ref[...], preferred_element_type=jnp.float32)
        emb = jnp.maximum(emb + bn_ref[...], 0.0)
        gi = jnp.dot(emb, wih_ref[...], preferred_element_type=jnp.float32)
        gi = gi + bih_ref[...]
        gh = jnp.dot(h, whh_ref[...], preferred_element_type=jnp.float32)
        gh = gh + bhh_ref[...]
        r = jax.nn.sigmoid(gi[:, :H] + gh[:, :H])
        z = jax.nn.sigmoid(gi[:, H:2 * H] + gh[:, H:2 * H])
        n = jnp.tanh(gi[:, 2 * H:] + r * gh[:, 2 * H:])
        h = (1.0 - z) * n + z * h
    o_ref[...] = h


def _gru_encode(xpad, p):
    rows = xpad.shape[0]
    blk = 1000
    grid = rows // blk
    full = lambda *_: (0, 0)
    return pl.pallas_call(
        _gru_body,
        grid=(grid,),
        in_specs=[
            pl.BlockSpec((blk, SEQ * OBS_IN), lambda i: (i, 0)),
            pl.BlockSpec((OBS_IN, H), full),
            pl.BlockSpec((1, H), full),
            pl.BlockSpec((H, 3 * H), full),
            pl.BlockSpec((H, 3 * H), full),
            pl.BlockSpec((1, 3 * H), full),
            pl.BlockSpec((1, 3 * H), full),
        ],
        out_specs=pl.BlockSpec((blk, H), lambda i: (i, 0)),
        out_shape=jax.ShapeDtypeStruct((rows, H), jnp.float32),
    )(xpad, p['nbrs_W'].T, p['nbrs_b'].reshape(1, H),
      p['W_ih'].T, p['W_hh'].T,
      p['b_ih'].reshape(1, 3 * H), p['b_hh'].reshape(1, 3 * H))


def _proj_body(x_ref, temb_ref, wl_ref, wr_ref, xl_ref, xr_ref, *, blk):
    c = pl.program_id(1)
    rows = jax.lax.broadcasted_iota(jnp.int32, (blk, H), 0) + c * blk
    t0 = jnp.broadcast_to(temb_ref[0:1, :], (blk, H))
    t1 = jnp.broadcast_to(temb_ref[1:2, :], (blk, H))
    tsel = jnp.where(rows >= NN, t1, t0)
    is_target = (rows == 0) | (rows == NN)
    x = jnp.where(is_target, tsel, x_ref[...])
    xl_ref[...] = jnp.dot(
        x, wl_ref[...], preferred_element_type=jnp.float32).astype(jnp.bfloat16)
    xr_ref[...] = jnp.dot(
        x, wr_ref[...], preferred_element_type=jnp.float32).astype(jnp.bfloat16)


def _project(x2, temb, p):
    blk = 2000
    nchunk = (BATCH * NN) // blk
    return pl.pallas_call(
        functools.partial(_proj_body, blk=blk),
        grid=(HEADS, nchunk),
        in_specs=[
            pl.BlockSpec((blk, H), lambda h, c: (c, 0)),
            pl.BlockSpec((BATCH, H), lambda h, c: (0, 0)),
            pl.BlockSpec((H, H), lambda h, c: (0, h)),
            pl.BlockSpec((H, H), lambda h, c: (0, h)),
        ],
        out_specs=[
            pl.BlockSpec((blk, H), lambda h, c: (c, h)),
            pl.BlockSpec((blk, H), lambda h, c: (c, h)),
        ],
        out_shape=[
            jax.ShapeDtypeStruct((TROWS, HEADS * H), jnp.bfloat16),
            jax.ShapeDtypeStruct((TROWS, HEADS * H), jnp.bfloat16),
        ],
    )(x2, temb, p['Wl'].T, p['Wr'].T)


def _node0_body(w_ref, x_ref, o_ref):
    o_ref[0] = jnp.dot(w_ref[0], x_ref[...].astype(jnp.float32),
                       preferred_element_type=jnp.float32)


def _node0_agg(w2, xlh):
    return pl.pallas_call(
        _node0_body,
        grid=(BATCH * HEADS,),
        in_specs=[
            pl.BlockSpec((1, 1, NN), lambda g: (g, 0, 0)),
            pl.BlockSpec((NN, H), lambda g: (g // HEADS, g % HEADS)),
        ],
        out_specs=pl.BlockSpec((1, 1, H), lambda g: (g, 0, 0)),
        out_shape=jax.ShapeDtypeStruct((BATCH * HEADS, 1, H), jnp.float32),
    )(w2, xlh)


def _head_body(o05_ref, m_ref, gatb_ref, hw_ref, hb_ref, vw_ref, vb_ref,
               aw_ref, ab_ref, q_ref):
    og = jnp.dot(m_ref[...], o05_ref[...],
                 preferred_element_type=jnp.float32) + gatb_ref[...]
    hid = jnp.dot(og, hw_ref[...], preferred_element_type=jnp.float32)
    hid = jnp.maximum(hid + hb_ref[...], 0.0)
    val = jnp.dot(hid, vw_ref[...], preferred_element_type=jnp.float32)
    val = val + vb_ref[...]
    adv = jnp.dot(hid, aw_ref[...], preferred_element_type=jnp.float32)
    adv = adv + ab_ref[...]
    q_ref[...] = val[:, 0:1] + adv - jnp.mean(adv, axis=-1, keepdims=True)


def _head(o05, p):
    mmix = jnp.zeros((BATCH, BATCH * HEADS), jnp.float32)
    rows = jnp.repeat(jnp.arange(BATCH), HEADS)
    cols = jnp.arange(BATCH * HEADS)
    mmix = mmix.at[rows, cols].set(1.0 / HEADS)
    vw = jnp.zeros((H, 8), jnp.float32).at[:, 0].set(p['out_W'][0])
    vb = jnp.zeros((1, 8), jnp.float32).at[0, 0].set(p['out_b'][0])
    return pl.pallas_call(
        _head_body,
        out_shape=jax.ShapeDtypeStruct((BATCH, ACTION), jnp.float32),
    )(o05, mmix, p['gat_b'].reshape(1, H), p['hid_W'].T,
      p['hid_b'].reshape(1, H), vw, vb, p['adv_W'].T,
      p['adv_b'].reshape(1, ACTION))


def _sc_edge_body(xlt, xrt, srcg, dstg, dstl, dstlg, srcw, att,
                  ex_hbm, den_hbm, atts_hbm, w_hbm,
                  xlb0, xlb1, xrb0, xrb1, exstage, exb2, dbuf, dbuf2,
                  wbuf, attsbuf,
                  isrcs, idsts, idstls, ibufb1, ibufb1b, ibufb2, ibufb2b,
                  attv, den_sp, w_sp,
                  semxl0, semxl1, semxr0, semxr1,
                  semd0, semd1, seme0, seme1,
                  ):
    c = lax.axis_index("c")
    s = lax.axis_index("s")
    iota = lax.iota(jnp.int32, LANES)
    zvec = jnp.zeros((LANES,), jnp.float32)
    mask5 = jnp.where(iota < HEADS, 1.0, 0.0).astype(jnp.float32)
    xlb = [xlb0, xlb1]
    xrb = [xrb0, xrb1]
    semxl = [semxl0, semxl1]
    semxr = [semxr0, semxr1]

    def zrow(i, carry):
        exstage[i, :] = zvec
        return carry
    lax.fori_loop(0, SUP, zrow, 0)
    for k in range(2):
        pltpu.sync_copy(exstage, den_sp.at[pl.ds(s * ZROWS + k * SUP, SUP)])
        pltpu.sync_copy(exstage, w_sp.at[pl.ds(s * ZROWS + k * SUP, SUP)])
    pltpu.sync_copy(exstage.at[pl.ds(0, ZROWS - 2 * SUP)],
                    den_sp.at[pl.ds(s * ZROWS + 2 * SUP, ZROWS - 2 * SUP)])
    pltpu.sync_copy(exstage.at[pl.ds(0, ZROWS - 2 * SUP)],
                    w_sp.at[pl.ds(s * ZROWS + 2 * SUP, ZROWS - 2 * SUP)])

    pltpu.sync_copy(att, attv)
    plsc.subcore_barrier()

    attregs = [[attv[h, pl.ds(dk * LANES, LANES)] for dk in range(H // LANES)]
               for h in range(HEADS)]

    ebase = c * EP + s * EPT

    def super_a(sc, carry):
        soff = ebase + sc * SUP
        pltpu.sync_copy(srcg.at[pl.ds(soff, SUP)], isrcs)
        pltpu.sync_copy(dstg.at[pl.ds(soff, SUP)], idsts)
        pltpu.sync_copy(dstl.at[pl.ds(soff, SUP)], idstls)
        pltpu.async_copy(xlt.at[isrcs.at[pl.ds(0, KCH)]], xlb[0], semxl[0])
        pltpu.async_copy(xrt.at[idsts.at[pl.ds(0, KCH)]], xrb[0], semxr[0])

        def pair_a(g2, pcarry):
            for b in range(2):
                g8 = g2 * 2 + b
                pltpu.make_async_copy(
                    xlt.at[isrcs.at[pl.ds(0, KCH)]], xlb[b],
                    semxl[b]).wait()
                pltpu.make_async_copy(
                    xrt.at[idsts.at[pl.ds(0, KCH)]], xrb[b],
                    semxr[b]).wait()

                @pl.when(g8 < CPS - 1)
                def _prefetch():
                    nx = (g8 + 1) * KCH
                    pltpu.async_copy(xlt.at[isrcs.at[pl.ds(nx, KCH)]],
                                     xlb[1 - b], semxl[1 - b])
                    pltpu.async_copy(xrt.at[idsts.at[pl.ds(nx, KCH)]],
                                     xrb[1 - b], semxr[1 - b])

                def edge_a(e, ecarry):
                    exrow = zvec
                    for h in range(HEADS):
                        acc = zvec
                        for dk in range(H // 32):
                            sl = pl.ds(h * H + dk * 32, 32)
                            t = xlb[b][e, sl] + xrb[b][e, sl]
                            t = jnp.maximum(t, t * 0.2)
                            te, to = plsc.unpack(t, format=plsc.PackFormat.INTERLEAVED)
                            acc = acc + te * attregs[h][2 * dk]
                            acc = acc + to * attregs[h][2 * dk + 1]
                        shv = jnp.full((LANES,), jnp.sum(acc), jnp.float32)
                        exrow = jnp.where(iota == h, shv, exrow)
                    exstage[g8 * KCH + e, :] = jnp.exp(exrow) * mask5
                    return ecarry
                lax.fori_loop(0, KCH, edge_a, 0)
            return pcarry
        lax.fori_loop(0, CPS // 2, pair_a, 0)

        pltpu.sync_copy(exstage, ex_hbm.at[pl.ds(soff, SUP)])
        pltpu.sync_copy(exstage, den_sp.at[idstls], add=True)
        return carry
    lax.fori_loop(0, NSUP, super_a, 0)

    plsc.subcore_barrier()
    pltpu.sync_copy(den_sp.at[pl.ds(s * ZROWS, ZROWS)],
                    den_hbm.at[pl.ds(c * NSEG + s * ZROWS, ZROWS)])
    plsc.subcore_barrier()

    exB = [exstage, exb2]
    dbufs = [dbuf, dbuf2]
    idx1 = [ibufb1, ibufb1b]
    idx2 = [ibufb2, ibufb2b]
    semd = [semd0, semd1]
    seme = [seme0, seme1]

    pltpu.sync_copy(ex_hbm.at[pl.ds(ebase, SUP)], exB[0])
    pltpu.sync_copy(dstlg.at[pl.ds(ebase, SUP)], idx1[0])
    pltpu.sync_copy(srcw.at[pl.ds(ebase, SUP)], idx2[0])
    pltpu.async_copy(den_hbm.at[idx1[0]], dbufs[0], semd[0])
    pltpu.async_copy(ex_hbm.at[pl.ds(ebase + SUP, SUP)], exB[1], seme[1])
    pltpu.async_copy(dstlg.at[pl.ds(ebase + SUP, SUP)], idx1[1], seme[1])
    pltpu.async_copy(srcw.at[pl.ds(ebase + SUP, SUP)], idx2[1], seme[1])

    def super_b2(sc2, carry):
        for b in range(2):
            sc = sc2 * 2 + b
            soff = ebase + sc * SUP
            pltpu.make_async_copy(den_hbm.at[idx1[b]], dbufs[b],
                                  semd[b]).wait()

            @pl.when(sc + 1 < NSUP)
            def _ready_next():
                noff = soff + SUP
                pltpu.make_async_copy(ex_hbm.at[pl.ds(noff, SUP)],
                                      exB[1 - b], seme[1 - b]).wait()
                pltpu.make_async_copy(dstlg.at[pl.ds(noff, SUP)],
                                      idx1[1 - b], seme[1 - b]).wait()
                pltpu.make_async_copy(srcw.at[pl.ds(noff, SUP)],
                                      idx2[1 - b], seme[1 - b]).wait()
                pltpu.async_copy(den_hbm.at[idx1[1 - b]], dbufs[1 - b],
                                 semd[1 - b])

            def group_b(j, gcarry):
                base = j * LANES
                attsvec = zvec
                for ee in range(LANES):
                    alpha = (exB[b][base + ee, :] /
                             (dbufs[b][base + ee, :] + 1e-16))
                    wbuf[base + ee, :] = alpha
                    sa = jnp.sum(alpha * mask5) * (1.0 / HEADS)
                    sav = jnp.full((LANES,), sa, jnp.float32)
                    attsvec = jnp.where(iota == ee, sav, attsvec)
                attsbuf[pl.ds(base, LANES)] = attsvec
                return gcarry
            lax.fori_loop(0, SUP // LANES, group_b, 0)

            pltpu.sync_copy(wbuf, w_sp.at[idx2[b]], add=True)
            pltpu.sync_copy(attsbuf, atts_hbm.at[pl.ds(soff, SUP)])

            @pl.when(sc + 2 < NSUP)
            def _issue_next2():
                noff2 = soff + 2 * SUP
                pltpu.async_copy(ex_hbm.at[pl.ds(noff2, SUP)], exB[b],
                                 seme[b])
                pltpu.async_copy(dstlg.at[pl.ds(noff2, SUP)], idx1[b],
                                 seme[b])
                pltpu.async_copy(srcw.at[pl.ds(noff2, SUP)], idx2[b],
                                 seme[b])
        return carry
    lax.fori_loop(0, NSUP // 2, super_b2, 0)

    plsc.subcore_barrier()
    pltpu.sync_copy(w_sp.at[pl.ds(s * ZROWS, ZROWS)],
                    w_hbm.at[pl.ds(c * NSEG + s * ZROWS, ZROWS)])


def _edge_stage(xlt, xrt, srcg, dstg, dstlg, dstl, srcw, att):
    mesh = plsc.VectorSubcoreMesh(core_axis_name="c", subcore_axis_name="s",
                                  num_cores=NC, num_subcores=NS)
    out_type = [
        jax.ShapeDtypeStruct((BATCH * EP, 16), jnp.float32),
        jax.ShapeDtypeStruct((BATCH * NSEG, 16), jnp.float32),
        jax.ShapeDtypeStruct((BATCH * EP,), jnp.float32),
        jax.ShapeDtypeStruct((BATCH * NSEG, 16), jnp.float32),
    ]
    scratch = [
        pltpu.VMEM((KCH, HEADS * H), jnp.bfloat16),
        pltpu.VMEM((KCH, HEADS * H), jnp.bfloat16),
        pltpu.VMEM((KCH, HEADS * H), jnp.bfloat16),
        pltpu.VMEM((KCH, HEADS * H), jnp.bfloat16),
        pltpu.VMEM((SUP, 16), jnp.float32),
        pltpu.VMEM((SUP, 16), jnp.float32),
        pltpu.VMEM((SUP, 16), jnp.float32),
        pltpu.VMEM((SUP, 16), jnp.float32),
        pltpu.VMEM((SUP, 16), jnp.float32),
        pltpu.VMEM((SUP,), jnp.float32),
        pltpu.VMEM((SUP,), jnp.int32),
        pltpu.VMEM((SUP,), jnp.int32),
        pltpu.VMEM((SUP,), jnp.int32),
        pltpu.VMEM((SUP,), jnp.int32),
        pltpu.VMEM((SUP,), jnp.int32),
        pltpu.VMEM((SUP,), jnp.int32),
        pltpu.VMEM((SUP,), jnp.int32),
        pltpu.VMEM((HEADS, H), jnp.float32),
        pltpu.VMEM_SHARED((NSEG, 16), jnp.float32),
        pltpu.VMEM_SHARED((NSEG, 16), jnp.float32),
    ] + [pltpu.SemaphoreType.DMA] * 8
    run = pl.kernel(_sc_edge_body, out_type=out_type, mesh=mesh,
                    scratch_types=scratch,
                    compiler_params=pltpu.CompilerParams(
                        use_tc_tiling_on_sc=False,
                        needs_layout_passes=False))
    return run(xlt, xrt, srcg, dstg, dstl, dstlg, srcw, att)


def _prep_edges(edges):
    ar = jnp.arange(NN, dtype=jnp.int32)
    src = jnp.concatenate(
        [edges[:, 0, :], jnp.broadcast_to(ar, (BATCH, NN))], axis=1)
    dst = jnp.concatenate(
        [edges[:, 1, :], jnp.broadcast_to(ar, (BATCH, NN))], axis=1)
    npad = EP - (NE + NN)
    srcp = jnp.pad(src, ((0, 0), (0, npad)))
    dstp = jnp.pad(dst, ((0, 0), (0, npad)))
    pad = (jnp.arange(EP) >= (NE + NN))[None, :]
    coff = (jnp.arange(BATCH, dtype=jnp.int32) * NN)[:, None]
    srcg = (srcp + coff).astype(jnp.int32)
    dstg = (dstp + coff).astype(jnp.int32)
    dst_loc = jnp.where(pad, TRASH, dstp).astype(jnp.int32)
    dstlg = dst_loc + (jnp.arange(BATCH, dtype=jnp.int32) * NSEG)[:, None]
    srcw = jnp.where((dstp == 0) & ~pad, srcp, TRASH).astype(jnp.int32)
    return (srcg.reshape(-1), dstg.reshape(-1), dstlg.reshape(-1),
            dst_loc.reshape(-1), srcw.reshape(-1))


def _cnn(obs_map, p):
    def patches(x, kh, kw, stride):
        return lax.conv_general_dilated_patches(
            x, (kh, kw), (stride, stride), 'VALID',
            dimension_numbers=('NCHW', 'OIHW', 'NCHW'))

    def conv(x, wkey, bkey, kh, kw, stride):
        w = p[wkey]
        co = w.shape[0]
        pt = patches(x, kh, kw, stride)
        n, f, ho, wo = pt.shape
        flat = pt.transpose(0, 2, 3, 1).reshape(n * ho * wo, f)
        y = _dense(flat, w.reshape(co, f).T, p[bkey], act=True)
        return y.reshape(n, ho, wo, co).transpose(0, 3, 1, 2)

    h = conv(obs_map, 'c1W', 'c1b', 8, 8, 4)
    h = conv(h, 'c2W', 'c2b', 4, 4, 2)
    h = conv(h, 'c3W', 'c3b', 3, 3, 1)
    flat = h.reshape(h.shape[0], -1)
    return _dense(flat, p['fcW'].T, p['fcb'], act=True)


@jax.jit
def kernel(obs, obs_map, edges, edges_feature, params):
    p = params
    target_emb = _cnn(obs_map, p)
    hid = _gru_encode(obs.reshape(BATCH * NN, SEQ * OBS_IN), p)
    xlt, xrt = _project(hid, target_emb, p)

    srcg, dstg, dstlg, dstl, srcw = _prep_edges(edges)
    att4 = p['att'].reshape(HEADS, H // 32, 16, 2)
    att_sc = att4.transpose(0, 1, 3, 2).reshape(HEADS, H)
    _, _, atts_full, w_full = _edge_stage(
        xlt, xrt, srcg, dstg, dstlg, dstl, srcw, att_sc)

    atts = atts_full.reshape(BATCH, EP)[:, :NE + 1]

    w2 = (w_full.reshape(BATCH, NSEG, 16)[:, :NN, :HEADS]
          .transpose(0, 2, 1).reshape(BATCH * HEADS, 1, NN))
    o05 = _node0_agg(w2, xlt).reshape(BATCH * HEADS, H)
    q = _head(o05, p)
    return q, atts


# speedup vs baseline: 1.0220x; 1.0220x over previous
import functools
import jax
import jax.numpy as jnp
from jax import lax
from jax.experimental import pallas as pl
from jax.experimental.pallas import tpu as pltpu
from jax.experimental.pallas import tpu_sc as plsc

H = 128
OBS_IN = 16
HEADS = 5
NN = 10000
NE = 160000
BATCH = 2
SEQ = 10
ACTION = 8

NC = 2
NS = 16
LANES = 16
KCH = 64
EP = 172032
EPT = EP // NS
NCHUNK = EPT // KCH
NSEG = 10240
TRASH = 10008
TROWS = BATCH * NN
ZROWS = NSEG // NS
SUP = 256
CPS = SUP // KCH
NSUP = EPT // SUP


def _dense_body(x_ref, w_ref, b_ref, o_ref, *, act):
    y = jnp.dot(x_ref[...], w_ref[...], preferred_element_type=jnp.float32)
    y = y + b_ref[...]
    if act:
        y = jnp.maximum(y, 0.0)
    o_ref[...] = y


def _dense(x, w, b, act=True):
    m, _ = x.shape
    n = w.shape[1]
    return pl.pallas_call(
        functools.partial(_dense_body, act=act),
        out_shape=jax.ShapeDtypeStruct((m, n), jnp.float32),
    )(x, w, b.reshape(1, n))


def _gru_body(x_ref, wn_ref, bn_ref, wih_ref, whh_ref, bih_ref, bhh_ref,
              o_ref):
    rows = x_ref.shape[0]
    h = jnp.zeros((rows, H), jnp.float32)
    for t in range(SEQ):
        xt = x_ref[:, t * OBS_IN:(t + 1) * OBS_IN]
        emb = jnp.dot(xt, wn_ref[...], preferred_element_type=jnp.float32)
        emb = jnp.maximum(emb + bn_ref[...], 0.0)
        gi = jnp.dot(emb, wih_ref[...], preferred_element_type=jnp.float32)
        gi = gi + bih_ref[...]
        gh = jnp.dot(h, whh_ref[...], preferred_element_type=jnp.float32)
        gh = gh + bhh_ref[...]
        r = jax.nn.sigmoid(gi[:, :H] + gh[:, :H])
        z = jax.nn.sigmoid(gi[:, H:2 * H] + gh[:, H:2 * H])
        n = jnp.tanh(gi[:, 2 * H:] + r * gh[:, 2 * H:])
        h = (1.0 - z) * n + z * h
    o_ref[...] = h


def _gru_encode(xpad, p):
    rows = xpad.shape[0]
    blk = 1000
    grid = rows // blk
    full = lambda *_: (0, 0)
    return pl.pallas_call(
        _gru_body,
        grid=(grid,),
        in_specs=[
            pl.BlockSpec((blk, SEQ * OBS_IN), lambda i: (i, 0)),
            pl.BlockSpec((OBS_IN, H), full),
            pl.BlockSpec((1, H), full),
            pl.BlockSpec((H, 3 * H), full),
            pl.BlockSpec((H, 3 * H), full),
            pl.BlockSpec((1, 3 * H), full),
            pl.BlockSpec((1, 3 * H), full),
        ],
        out_specs=pl.BlockSpec((blk, H), lambda i: (i, 0)),
        out_shape=jax.ShapeDtypeStruct((rows, H), jnp.float32),
    )(xpad, p['nbrs_W'].T, p['nbrs_b'].reshape(1, H),
      p['W_ih'].T, p['W_hh'].T,
      p['b_ih'].reshape(1, 3 * H), p['b_hh'].reshape(1, 3 * H))


def _proj_body(x_ref, temb_ref, wl_ref, wr_ref, xl_ref, xr_ref, *, blk):
    c = pl.program_id(1)
    rows = jax.lax.broadcasted_iota(jnp.int32, (blk, H), 0) + c * blk
    t0 = jnp.broadcast_to(temb_ref[0:1, :], (blk, H))
    t1 = jnp.broadcast_to(temb_ref[1:2, :], (blk, H))
    tsel = jnp.where(rows >= NN, t1, t0)
    is_target = (rows == 0) | (rows == NN)
    x = jnp.where(is_target, tsel, x_ref[...])
    xl_ref[...] = jnp.dot(
        x, wl_ref[...], preferred_element_type=jnp.float32).astype(jnp.bfloat16)
    xr_ref[...] = jnp.dot(
        x, wr_ref[...], preferred_element_type=jnp.float32).astype(jnp.bfloat16)


def _project(x2, temb, p):
    blk = 2000
    nchunk = (BATCH * NN) // blk
    return pl.pallas_call(
        functools.partial(_proj_body, blk=blk),
        grid=(HEADS, nchunk),
        in_specs=[
            pl.BlockSpec((blk, H), lambda h, c: (c, 0)),
            pl.BlockSpec((BATCH, H), lambda h, c: (0, 0)),
            pl.BlockSpec((H, H), lambda h, c: (0, h)),
            pl.BlockSpec((H, H), lambda h, c: (0, h)),
        ],
        out_specs=[
            pl.BlockSpec((blk, H), lambda h, c: (c, h)),
            pl.BlockSpec((blk, H), lambda h, c: (c, h)),
        ],
        out_shape=[
            jax.ShapeDtypeStruct((TROWS, HEADS * H), jnp.bfloat16),
            jax.ShapeDtypeStruct((TROWS, HEADS * H), jnp.bfloat16),
        ],
    )(x2, temb, p['Wl'].T, p['Wr'].T)


def _node0_body(w_ref, x_ref, o_ref):
    o_ref[0] = jnp.dot(w_ref[0], x_ref[...].astype(jnp.float32),
                       preferred_element_type=jnp.float32)


def _node0_agg(w2, xlh):
    return pl.pallas_call(
        _node0_body,
        grid=(BATCH * HEADS,),
        in_specs=[
            pl.BlockSpec((1, 1, NN), lambda g: (g, 0, 0)),
            pl.BlockSpec((NN, H), lambda g: (g // HEADS, g % HEADS)),
        ],
        out_specs=pl.BlockSpec((1, 1, H), lambda g: (g, 0, 0)),
        out_shape=jax.ShapeDtypeStruct((BATCH * HEADS, 1, H), jnp.float32),
    )(w2, xlh)


def _head_body(o05_ref, m_ref, gatb_ref, hw_ref, hb_ref, vw_ref, vb_ref,
               aw_ref, ab_ref, q_ref):
    og = jnp.dot(m_ref[...], o05_ref[...],
                 preferred_element_type=jnp.float32) + gatb_ref[...]
    hid = jnp.dot(og, hw_ref[...], preferred_element_type=jnp.float32)
    hid = jnp.maximum(hid + hb_ref[...], 0.0)
    val = jnp.dot(hid, vw_ref[...], preferred_element_type=jnp.float32)
    val = val + vb_ref[...]
    adv = jnp.dot(hid, aw_ref[...], preferred_element_type=jnp.float32)
    adv = adv + ab_ref[...]
    q_ref[...] = val[:, 0:1] + adv - jnp.mean(adv, axis=-1, keepdims=True)


def _head(o05, p):
    mmix = jnp.zeros((BATCH, BATCH * HEADS), jnp.float32)
    rows = jnp.repeat(jnp.arange(BATCH), HEADS)
    cols = jnp.arange(BATCH * HEADS)
    mmix = mmix.at[rows, cols].set(1.0 / HEADS)
    vw = jnp.zeros((H, 8), jnp.float32).at[:, 0].set(p['out_W'][0])
    vb = jnp.zeros((1, 8), jnp.float32).at[0, 0].set(p['out_b'][0])
    return pl.pallas_call(
        _head_body,
        out_shape=jax.ShapeDtypeStruct((BATCH, ACTION), jnp.float32),
    )(o05, mmix, p['gat_b'].reshape(1, H), p['hid_W'].T,
      p['hid_b'].reshape(1, H), vw, vb, p['adv_W'].T,
      p['adv_b'].reshape(1, ACTION))


def _sc_edge_body(xlt, xrt, srcg, dstg, dstl, dstlg, srcw, att,
                  ex_hbm, den_hbm, atts_hbm, w_hbm,
                  xlb0, xlb1, xrb0, xrb1, exstage, exb2, dbuf, dbuf2,
                  wbuf, attsbuf,
                  isrcs, idsts, idstls, ibufb1, ibufb1b, ibufb2, ibufb2b,
                  attv, den_sp, w_sp,
                  semxl0, semxl1, semxr0, semxr1,
                  semd0, semd1, seme0, seme1,
                  ):
    c = lax.axis_index("c")
    s = lax.axis_index("s")
    iota = lax.iota(jnp.int32, LANES)
    zvec = jnp.zeros((LANES,), jnp.float32)
    mask5 = jnp.where(iota < HEADS, 1.0, 0.0).astype(jnp.float32)
    xlb = [xlb0, xlb1]
    xrb = [xrb0, xrb1]
    semxl = [semxl0, semxl1]
    semxr = [semxr0, semxr1]

    def zrow(i, carry):
        exstage[i, :] = zvec
        return carry
    lax.fori_loop(0, SUP, zrow, 0)
    for k in range(2):
        pltpu.sync_copy(exstage, den_sp.at[pl.ds(s * ZROWS + k * SUP, SUP)])
        pltpu.sync_copy(exstage, w_sp.at[pl.ds(s * ZROWS + k * SUP, SUP)])
    pltpu.sync_copy(exstage.at[pl.ds(0, ZROWS - 2 * SUP)],
                    den_sp.at[pl.ds(s * ZROWS + 2 * SUP, ZROWS - 2 * SUP)])
    pltpu.sync_copy(exstage.at[pl.ds(0, ZROWS - 2 * SUP)],
                    w_sp.at[pl.ds(s * ZROWS + 2 * SUP, ZROWS - 2 * SUP)])

    pltpu.sync_copy(att, attv)
    plsc.subcore_barrier()

    attregs = [[attv[h, pl.ds(dk * LANES, LANES)] for dk in range(H // LANES)]
               for h in range(HEADS)]

    ebase = c * EP + s * EPT

    pltpu.sync_copy(srcg.at[pl.ds(ebase, SUP)], isrcs)
    pltpu.sync_copy(dstg.at[pl.ds(ebase, SUP)], idsts)
    pltpu.async_copy(xlt.at[isrcs.at[pl.ds(0, KCH)]], xlb[0], semxl[0])
    pltpu.async_copy(xrt.at[idsts.at[pl.ds(0, KCH)]], xrb[0], semxr[0])

    def super_a(sc, carry):
        soff = ebase + sc * SUP
        pltpu.sync_copy(dstl.at[pl.ds(soff, SUP)], idstls)

        def pair_a(g2, pcarry):
            for b in range(2):
                g8 = g2 * 2 + b
                pltpu.make_async_copy(
                    xlt.at[isrcs.at[pl.ds(0, KCH)]], xlb[b],
                    semxl[b]).wait()
                pltpu.make_async_copy(
                    xrt.at[idsts.at[pl.ds(0, KCH)]], xrb[b],
                    semxr[b]).wait()

                @pl.when(g8 < CPS - 1)
                def _prefetch():
                    nx = (g8 + 1) * KCH
                    pltpu.async_copy(xlt.at[isrcs.at[pl.ds(nx, KCH)]],
                                     xlb[1 - b], semxl[1 - b])
                    pltpu.async_copy(xrt.at[idsts.at[pl.ds(nx, KCH)]],
                                     xrb[1 - b], semxr[1 - b])

                def edge_a(e, ecarry):
                    exrow = zvec
                    for h in range(HEADS):
                        acc = zvec
                        for dk in range(H // 32):
                            sl = pl.ds(h * H + dk * 32, 32)
                            t = xlb[b][e, sl] + xrb[b][e, sl]
                            t = jnp.maximum(t, t * 0.2)
                            te, to = plsc.unpack(t, format=plsc.PackFormat.INTERLEAVED)
                            acc = acc + te * attregs[h][2 * dk]
                            acc = acc + to * attregs[h][2 * dk + 1]
                        shv = jnp.full((LANES,), jnp.sum(acc), jnp.float32)
                        exrow = jnp.where(iota == h, shv, exrow)
                    exstage[g8 * KCH + e, :] = jnp.exp(exrow) * mask5
                    return ecarry
                lax.fori_loop(0, KCH, edge_a, 0)
            return pcarry
        lax.fori_loop(0, CPS // 2, pair_a, 0)

        @pl.when(sc + 1 < NSUP)
        def _next_super():
            noff = soff + SUP
            pltpu.sync_copy(srcg.at[pl.ds(noff, SUP)], isrcs)
            pltpu.sync_copy(dstg.at[pl.ds(noff, SUP)], idsts)
            pltpu.async_copy(xlt.at[isrcs.at[pl.ds(0, KCH)]], xlb[0],
                             semxl[0])
            pltpu.async_copy(xrt.at[idsts.at[pl.ds(0, KCH)]], xrb[0],
                             semxr[0])

        pltpu.sync_copy(exstage, ex_hbm.at[pl.ds(soff, SUP)])
        pltpu.sync_copy(exstage, den_sp.at[idstls], add=True)
        return carry
    lax.fori_loop(0, NSUP, super_a, 0)

    plsc.subcore_barrier()
    pltpu.sync_copy(den_sp.at[pl.ds(s * ZROWS, ZROWS)],
                    den_hbm.at[pl.ds(c * NSEG + s * ZROWS, ZROWS)])
    plsc.subcore_barrier()

    exB = [exstage, exb2]
    dbufs = [dbuf, dbuf2]
    idx1 = [ibufb1, ibufb1b]
    idx2 = [ibufb2, ibufb2b]
    semd = [semd0, semd1]
    seme = [seme0, seme1]

    pltpu.sync_copy(ex_hbm.at[pl.ds(ebase, SUP)], exB[0])
    pltpu.sync_copy(dstlg.at[pl.ds(ebase, SUP)], idx1[0])
    pltpu.sync_copy(srcw.at[pl.ds(ebase, SUP)], idx2[0])
    pltpu.async_copy(den_hbm.at[idx1[0]], dbufs[0], semd[0])
    pltpu.async_copy(ex_hbm.at[pl.ds(ebase + SUP, SUP)], exB[1], seme[1])
    pltpu.async_copy(dstlg.at[pl.ds(ebase + SUP, SUP)], idx1[1], seme[1])
    pltpu.async_copy(srcw.at[pl.ds(ebase + SUP, SUP)], idx2[1], seme[1])

    def super_b2(sc2, carry):
        for b in range(2):
            sc = sc2 * 2 + b
            soff = ebase + sc * SUP
            pltpu.make_async_copy(den_hbm.at[idx1[b]], dbufs[b],
                                  semd[b]).wait()

            @pl.when(sc + 1 < NSUP)
            def _ready_next():
                noff = soff + SUP
                pltpu.make_async_copy(ex_hbm.at[pl.ds(noff, SUP)],
                                      exB[1 - b], seme[1 - b]).wait()
                pltpu.make_async_copy(dstlg.at[pl.ds(noff, SUP)],
                                      idx1[1 - b], seme[1 - b]).wait()
                pltpu.make_async_copy(srcw.at[pl.ds(noff, SUP)],
                                      idx2[1 - b], seme[1 - b]).wait()
                pltpu.async_copy(den_hbm.at[idx1[1 - b]], dbufs[1 - b],
                                 semd[1 - b])

            def group_b(j, gcarry):
                base = j * LANES
                attsvec = zvec
                for ee in range(LANES):
                    alpha = (exB[b][base + ee, :] /
                             (dbufs[b][base + ee, :] + 1e-16))
                    wbuf[base + ee, :] = alpha
                    sa = jnp.sum(alpha * mask5) * (1.0 / HEADS)
                    sav = jnp.full((LANES,), sa, jnp.float32)
                    attsvec = jnp.where(iota == ee, sav, attsvec)
                attsbuf[pl.ds(base, LANES)] = attsvec
                return gcarry
            lax.fori_loop(0, SUP // LANES, group_b, 0)

            pltpu.sync_copy(wbuf, w_sp.at[idx2[b]], add=True)
            pltpu.sync_copy(attsbuf, atts_hbm.at[pl.ds(soff, SUP)])

            @pl.when(sc + 2 < NSUP)
            def _issue_next2():
                noff2 = soff + 2 * SUP
                pltpu.async_copy(ex_hbm.at[pl.ds(noff2, SUP)], exB[b],
                                 seme[b])
                pltpu.async_copy(dstlg.at[pl.ds(noff2, SUP)], idx1[b],
                                 seme[b])
                pltpu.async_copy(srcw.at[pl.ds(noff2, SUP)], idx2[b],
                                 seme[b])
        return carry
    lax.fori_loop(0, NSUP // 2, super_b2, 0)

    plsc.subcore_barrier()
    pltpu.sync_copy(w_sp.at[pl.ds(s * ZROWS, ZROWS)],
                    w_hbm.at[pl.ds(c * NSEG + s * ZROWS, ZROWS)])


def _edge_stage(xlt, xrt, srcg, dstg, dstlg, dstl, srcw, att):
    mesh = plsc.VectorSubcoreMesh(core_axis_name="c", subcore_axis_name="s",
                                  num_cores=NC, num_subcores=NS)
    out_type = [
        jax.ShapeDtypeStruct((BATCH * EP, 16), jnp.float32),
        jax.ShapeDtypeStruct((BATCH * NSEG, 16), jnp.float32),
        jax.ShapeDtypeStruct((BATCH * EP,), jnp.float32),
        jax.ShapeDtypeStruct((BATCH * NSEG, 16), jnp.float32),
    ]
    scratch = [
        pltpu.VMEM((KCH, HEADS * H), jnp.bfloat16),
        pltpu.VMEM((KCH, HEADS * H), jnp.bfloat16),
        pltpu.VMEM((KCH, HEADS * H), jnp.bfloat16),
        pltpu.VMEM((KCH, HEADS * H), jnp.bfloat16),
        pltpu.VMEM((SUP, 16), jnp.float32),
        pltpu.VMEM((SUP, 16), jnp.float32),
        pltpu.VMEM((SUP, 16), jnp.float32),
        pltpu.VMEM((SUP, 16), jnp.float32),
        pltpu.VMEM((SUP, 16), jnp.float32),
        pltpu.VMEM((SUP,), jnp.float32),
        pltpu.VMEM((SUP,), jnp.int32),
        pltpu.VMEM((SUP,), jnp.int32),
        pltpu.VMEM((SUP,), jnp.int32),
        pltpu.VMEM((SUP,), jnp.int32),
        pltpu.VMEM((SUP,), jnp.int32),
        pltpu.VMEM((SUP,), jnp.int32),
        pltpu.VMEM((SUP,), jnp.int32),
        pltpu.VMEM((HEADS, H), jnp.float32),
        pltpu.VMEM_SHARED((NSEG, 16), jnp.float32),
        pltpu.VMEM_SHARED((NSEG, 16), jnp.float32),
    ] + [pltpu.SemaphoreType.DMA] * 8
    run = pl.kernel(_sc_edge_body, out_type=out_type, mesh=mesh,
                    scratch_types=scratch,
                    compiler_params=pltpu.CompilerParams(
                        use_tc_tiling_on_sc=False,
                        needs_layout_passes=False))
    return run(xlt, xrt, srcg, dstg, dstl, dstlg, srcw, att)


def _prep_edges(edges):
    ar = jnp.arange(NN, dtype=jnp.int32)
    src = jnp.concatenate(
        [edges[:, 0, :], jnp.broadcast_to(ar, (BATCH, NN))], axis=1)
    dst = jnp.concatenate(
        [edges[:, 1, :], jnp.broadcast_to(ar, (BATCH, NN))], axis=1)
    npad = EP - (NE + NN)
    srcp = jnp.pad(src, ((0, 0), (0, npad)))
    dstp = jnp.pad(dst, ((0, 0), (0, npad)))
    pad = (jnp.arange(EP) >= (NE + NN))[None, :]
    coff = (jnp.arange(BATCH, dtype=jnp.int32) * NN)[:, None]
    srcg = (srcp + coff).astype(jnp.int32)
    dstg = (dstp + coff).astype(jnp.int32)
    dst_loc = jnp.where(pad, TRASH, dstp).astype(jnp.int32)
    dstlg = dst_loc + (jnp.arange(BATCH, dtype=jnp.int32) * NSEG)[:, None]
    srcw = jnp.where((dstp == 0) & ~pad, srcp, TRASH).astype(jnp.int32)
    return (srcg.reshape(-1), dstg.reshape(-1), dstlg.reshape(-1),
            dst_loc.reshape(-1), srcw.reshape(-1))


def _cnn(obs_map, p):
    def patches(x, kh, kw, stride):
        return lax.conv_general_dilated_patches(
            x, (kh, kw), (stride, stride), 'VALID',
            dimension_numbers=('NCHW', 'OIHW', 'NCHW'))

    def conv(x, wkey, bkey, kh, kw, stride):
        w = p[wkey]
        co = w.shape[0]
        pt = patches(x, kh, kw, stride)
        n, f, ho, wo = pt.shape
        flat = pt.transpose(0, 2, 3, 1).reshape(n * ho * wo, f)
        y = _dense(flat, w.reshape(co, f).T, p[bkey], act=True)
        return y.reshape(n, ho, wo, co).transpose(0, 3, 1, 2)

    h = conv(obs_map, 'c1W', 'c1b', 8, 8, 4)
    h = conv(h, 'c2W', 'c2b', 4, 4, 2)
    h = conv(h, 'c3W', 'c3b', 3, 3, 1)
    flat = h.reshape(h.shape[0], -1)
    return _dense(flat, p['fcW'].T, p['fcb'], act=True)


@jax.jit
def kernel(obs, obs_map, edges, edges_feature, params):
    p = params
    target_emb = _cnn(obs_map, p)
    hid = _gru_encode(obs.reshape(BATCH * NN, SEQ * OBS_IN), p)
    xlt, xrt = _project(hid, target_emb, p)

    srcg, dstg, dstlg, dstl, srcw = _prep_edges(edges)
    att4 = p['att'].reshape(HEADS, H // 32, 16, 2)
    att_sc = att4.transpose(0, 1, 3, 2).reshape(HEADS, H)
    _, _, atts_full, w_full = _edge_stage(
        xlt, xrt, srcg, dstg, dstlg, dstl, srcw, att_sc)

    atts = atts_full.reshape(BATCH, EP)[:, :NE + 1]

    w2 = (w_full.reshape(BATCH, NSEG, 16)[:, :NN, :HEADS]
          .transpose(0, 2, 1).reshape(BATCH * HEADS, 1, NN))
    o05 = _node0_agg(w2, xlt).reshape(BATCH * HEADS, H)
    q = _head(o05, p)
    return q, atts


# speedup vs baseline: 1.0254x; 1.0033x over previous
import functools
import jax
import jax.numpy as jnp
from jax import lax
from jax.experimental import pallas as pl
from jax.experimental.pallas import tpu as pltpu
from jax.experimental.pallas import tpu_sc as plsc

H = 128
OBS_IN = 16
HEADS = 5
NN = 10000
NE = 160000
BATCH = 2
SEQ = 10
ACTION = 8

NC = 2
NS = 16
LANES = 16
KCH = 64
EP = 172032
EPT = EP // NS
NCHUNK = EPT // KCH
NSEG = 10240
TRASH = 10008
TROWS = BATCH * NN
ZROWS = NSEG // NS
SUP = 256
CPS = SUP // KCH
NSUP = EPT // SUP


def _dense_body(x_ref, w_ref, b_ref, o_ref, *, act):
    y = jnp.dot(x_ref[...], w_ref[...], preferred_element_type=jnp.float32)
    y = y + b_ref[...]
    if act:
        y = jnp.maximum(y, 0.0)
    o_ref[...] = y


def _dense(x, w, b, act=True):
    m, _ = x.shape
    n = w.shape[1]
    return pl.pallas_call(
        functools.partial(_dense_body, act=act),
        out_shape=jax.ShapeDtypeStruct((m, n), jnp.float32),
    )(x, w, b.reshape(1, n))


def _gru_body(x_ref, wn_ref, bn_ref, wih_ref, whh_ref, bih_ref, bhh_ref,
              o_ref):
    rows = x_ref.shape[0]
    h = jnp.zeros((rows, H), jnp.float32)
    for t in range(SEQ):
        xt = x_ref[:, t * OBS_IN:(t + 1) * OBS_IN]
        emb = jnp.dot(xt, wn_ref[...], preferred_element_type=jnp.float32)
        emb = jnp.maximum(emb + bn_ref[...], 0.0)
        gi = jnp.dot(emb, wih_ref[...], preferred_element_type=jnp.float32)
        gi = gi + bih_ref[...]
        gh = jnp.dot(h, whh_ref[...], preferred_element_type=jnp.float32)
        gh = gh + bhh_ref[...]
        r = jax.nn.sigmoid(gi[:, :H] + gh[:, :H])
        z = jax.nn.sigmoid(gi[:, H:2 * H] + gh[:, H:2 * H])
        n = jnp.tanh(gi[:, 2 * H:] + r * gh[:, 2 * H:])
        h = (1.0 - z) * n + z * h
    o_ref[...] = h


def _gru_encode(xpad, p):
    rows = xpad.shape[0]
    blk = 1000
    grid = rows // blk
    full = lambda *_: (0, 0)
    return pl.pallas_call(
        _gru_body,
        grid=(grid,),
        in_specs=[
            pl.BlockSpec((blk, SEQ * OBS_IN), lambda i: (i, 0)),
            pl.BlockSpec((OBS_IN, H), full),
            pl.BlockSpec((1, H), full),
            pl.BlockSpec((H, 3 * H), full),
            pl.BlockSpec((H, 3 * H), full),
            pl.BlockSpec((1, 3 * H), full),
            pl.BlockSpec((1, 3 * H), full),
        ],
        out_specs=pl.BlockSpec((blk, H), lambda i: (i, 0)),
        out_shape=jax.ShapeDtypeStruct((rows, H), jnp.float32),
    )(xpad, p['nbrs_W'].T, p['nbrs_b'].reshape(1, H),
      p['W_ih'].T, p['W_hh'].T,
      p['b_ih'].reshape(1, 3 * H), p['b_hh'].reshape(1, 3 * H))


def _proj_body(x_ref, temb_ref, wl_ref, wr_ref, xl_ref, xr_ref, *, blk):
    c = pl.program_id(1)
    rows = jax.lax.broadcasted_iota(jnp.int32, (blk, H), 0) + c * blk
    t0 = jnp.broadcast_to(temb_ref[0:1, :], (blk, H))
    t1 = jnp.broadcast_to(temb_ref[1:2, :], (blk, H))
    tsel = jnp.where(rows >= NN, t1, t0)
    is_target = (rows == 0) | (rows == NN)
    x = jnp.where(is_target, tsel, x_ref[...])
    xl_ref[...] = jnp.dot(
        x, wl_ref[...], preferred_element_type=jnp.float32).astype(jnp.bfloat16)
    xr_ref[...] = jnp.dot(
        x, wr_ref[...], preferred_element_type=jnp.float32).astype(jnp.bfloat16)


def _project(x2, temb, p):
    blk = 2000
    nchunk = (BATCH * NN) // blk
    return pl.pallas_call(
        functools.partial(_proj_body, blk=blk),
        grid=(HEADS, nchunk),
        in_specs=[
            pl.BlockSpec((blk, H), lambda h, c: (c, 0)),
            pl.BlockSpec((BATCH, H), lambda h, c: (0, 0)),
            pl.BlockSpec((H, H), lambda h, c: (0, h)),
            pl.BlockSpec((H, H), lambda h, c: (0, h)),
        ],
        out_specs=[
            pl.BlockSpec((blk, H), lambda h, c: (c, h)),
            pl.BlockSpec((blk, H), lambda h, c: (c, h)),
        ],
        out_shape=[
            jax.ShapeDtypeStruct((TROWS, HEADS * H), jnp.bfloat16),
            jax.ShapeDtypeStruct((TROWS, HEADS * H), jnp.bfloat16),
        ],
    )(x2, temb, p['Wl'].T, p['Wr'].T)


def _node0_body(w_ref, x_ref, o_ref):
    o_ref[0] = jnp.dot(w_ref[0], x_ref[...].astype(jnp.float32),
                       preferred_element_type=jnp.float32)


def _node0_agg(w2, xlh):
    return pl.pallas_call(
        _node0_body,
        grid=(BATCH * HEADS,),
        in_specs=[
            pl.BlockSpec((1, 1, NN), lambda g: (g, 0, 0)),
            pl.BlockSpec((NN, H), lambda g: (g // HEADS, g % HEADS)),
        ],
        out_specs=pl.BlockSpec((1, 1, H), lambda g: (g, 0, 0)),
        out_shape=jax.ShapeDtypeStruct((BATCH * HEADS, 1, H), jnp.float32),
    )(w2, xlh)


def _head_body(o05_ref, m_ref, gatb_ref, hw_ref, hb_ref, vw_ref, vb_ref,
               aw_ref, ab_ref, q_ref):
    og = jnp.dot(m_ref[...], o05_ref[...],
                 preferred_element_type=jnp.float32) + gatb_ref[...]
    hid = jnp.dot(og, hw_ref[...], preferred_element_type=jnp.float32)
    hid = jnp.maximum(hid + hb_ref[...], 0.0)
    val = jnp.dot(hid, vw_ref[...], preferred_element_type=jnp.float32)
    val = val + vb_ref[...]
    adv = jnp.dot(hid, aw_ref[...], preferred_element_type=jnp.float32)
    adv = adv + ab_ref[...]
    q_ref[...] = val[:, 0:1] + adv - jnp.mean(adv, axis=-1, keepdims=True)


def _head(o05, p):
    mmix = jnp.zeros((BATCH, BATCH * HEADS), jnp.float32)
    rows = jnp.repeat(jnp.arange(BATCH), HEADS)
    cols = jnp.arange(BATCH * HEADS)
    mmix = mmix.at[rows, cols].set(1.0 / HEADS)
    vw = jnp.zeros((H, 8), jnp.float32).at[:, 0].set(p['out_W'][0])
    vb = jnp.zeros((1, 8), jnp.float32).at[0, 0].set(p['out_b'][0])
    return pl.pallas_call(
        _head_body,
        out_shape=jax.ShapeDtypeStruct((BATCH, ACTION), jnp.float32),
    )(o05, mmix, p['gat_b'].reshape(1, H), p['hid_W'].T,
      p['hid_b'].reshape(1, H), vw, vb, p['adv_W'].T,
      p['adv_b'].reshape(1, ACTION))


def _sc_edge_body(xlt, xrt, srcg, dstg, dstl, dstlg, srcw, att,
                  ex_hbm, den_hbm, atts_hbm, w_hbm,
                  xlb0, xlb1, xrb0, xrb1, exstage, exb2, dbuf, dbuf2,
                  wbuf, attsbuf,
                  isrcs, idsts, idstls, ibufb1, ibufb1b, ibufb2, ibufb2b,
                  attv, den_sp, w_sp,
                  semxl0, semxl1, semxr0, semxr1,
                  semd0, semd1, seme0, seme1,
                  ):
    c = lax.axis_index("c")
    s = lax.axis_index("s")
    iota = lax.iota(jnp.int32, LANES)
    zvec = jnp.zeros((LANES,), jnp.float32)
    mask5 = jnp.where(iota < HEADS, 1.0, 0.0).astype(jnp.float32)
    xlb = [xlb0, xlb1]
    xrb = [xrb0, xrb1]
    semxl = [semxl0, semxl1]
    semxr = [semxr0, semxr1]

    def zrow(i, carry):
        exstage[i, :] = zvec
        return carry
    lax.fori_loop(0, SUP, zrow, 0)
    for k in range(2):
        pltpu.sync_copy(exstage, den_sp.at[pl.ds(s * ZROWS + k * SUP, SUP)])
        pltpu.sync_copy(exstage, w_sp.at[pl.ds(s * ZROWS + k * SUP, SUP)])
    pltpu.sync_copy(exstage.at[pl.ds(0, ZROWS - 2 * SUP)],
                    den_sp.at[pl.ds(s * ZROWS + 2 * SUP, ZROWS - 2 * SUP)])
    pltpu.sync_copy(exstage.at[pl.ds(0, ZROWS - 2 * SUP)],
                    w_sp.at[pl.ds(s * ZROWS + 2 * SUP, ZROWS - 2 * SUP)])

    pltpu.sync_copy(att, attv)
    plsc.subcore_barrier()

    attregs = [[attv[h, pl.ds(dk * LANES, LANES)] for dk in range(H // LANES)]
               for h in range(HEADS)]

    ebase = c * EP + s * EPT

    pltpu.sync_copy(srcg.at[pl.ds(ebase, SUP)], isrcs)
    pltpu.sync_copy(dstg.at[pl.ds(ebase, SUP)], idsts)
    pltpu.async_copy(xlt.at[isrcs.at[pl.ds(0, KCH)]], xlb[0], semxl[0])
    pltpu.async_copy(xrt.at[idsts.at[pl.ds(0, KCH)]], xrb[0], semxr[0])

    stages = [exstage, exb2]
    idls = [idstls, ibufb1]
    semw = [semd0, semd1]
    semsc = [seme0, seme1]

    def super_pair_a(sc2, carry):
        for sb in range(2):
            sc = sc2 * 2 + sb
            stage = stages[sb]
            soff = ebase + sc * SUP

            @pl.when(sc >= 2)
            def _drain_stage():
                poff = soff - 2 * SUP
                pltpu.make_async_copy(stage, ex_hbm.at[pl.ds(poff, SUP)],
                                      semw[sb]).wait()
                pltpu.make_async_copy(stage, den_sp.at[idls[sb]],
                                      semsc[sb]).wait()

            pltpu.sync_copy(dstl.at[pl.ds(soff, SUP)], idls[sb])

            def pair_a(g2, pcarry):
                for b in range(2):
                    g8 = g2 * 2 + b
                    pltpu.make_async_copy(
                        xlt.at[isrcs.at[pl.ds(0, KCH)]], xlb[b],
                        semxl[b]).wait()
                    pltpu.make_async_copy(
                        xrt.at[idsts.at[pl.ds(0, KCH)]], xrb[b],
                        semxr[b]).wait()

                    @pl.when(g8 < CPS - 1)
                    def _prefetch():
                        nx = (g8 + 1) * KCH
                        pltpu.async_copy(xlt.at[isrcs.at[pl.ds(nx, KCH)]],
                                         xlb[1 - b], semxl[1 - b])
                        pltpu.async_copy(xrt.at[idsts.at[pl.ds(nx, KCH)]],
                                         xrb[1 - b], semxr[1 - b])

                    def edge_a(e, ecarry):
                        exrow = zvec
                        for h in range(HEADS):
                            acc = zvec
                            for dk in range(H // 32):
                                sl = pl.ds(h * H + dk * 32, 32)
                                t = xlb[b][e, sl] + xrb[b][e, sl]
                                t = jnp.maximum(t, t * 0.2)
                                te, to = plsc.unpack(
                                    t, format=plsc.PackFormat.INTERLEAVED)
                                acc = acc + te * attregs[h][2 * dk]
                                acc = acc + to * attregs[h][2 * dk + 1]
                            shv = jnp.full((LANES,), jnp.sum(acc),
                                           jnp.float32)
                            exrow = jnp.where(iota == h, shv, exrow)
                        stage[g8 * KCH + e, :] = jnp.exp(exrow) * mask5
                        return ecarry
                    lax.fori_loop(0, KCH, edge_a, 0)
                return pcarry
            lax.fori_loop(0, CPS // 2, pair_a, 0)

            @pl.when(sc + 1 < NSUP)
            def _next_super():
                noff = soff + SUP
                pltpu.sync_copy(srcg.at[pl.ds(noff, SUP)], isrcs)
                pltpu.sync_copy(dstg.at[pl.ds(noff, SUP)], idsts)
                pltpu.async_copy(xlt.at[isrcs.at[pl.ds(0, KCH)]], xlb[0],
                                 semxl[0])
                pltpu.async_copy(xrt.at[idsts.at[pl.ds(0, KCH)]], xrb[0],
                                 semxr[0])

            pltpu.async_copy(stage, ex_hbm.at[pl.ds(soff, SUP)], semw[sb])
            pltpu.async_copy(stage, den_sp.at[idls[sb]], semsc[sb],
                             add=True)
        return carry
    lax.fori_loop(0, NSUP // 2, super_pair_a, 0)

    for sb in range(2):
        poff = ebase + (NSUP - 2 + sb) * SUP
        pltpu.make_async_copy(stages[sb], ex_hbm.at[pl.ds(poff, SUP)],
                              semw[sb]).wait()
        pltpu.make_async_copy(stages[sb], den_sp.at[idls[sb]],
                              semsc[sb]).wait()

    plsc.subcore_barrier()
    pltpu.sync_copy(den_sp.at[pl.ds(s * ZROWS, ZROWS)],
                    den_hbm.at[pl.ds(c * NSEG + s * ZROWS, ZROWS)])
    plsc.subcore_barrier()

    exB = [exstage, exb2]
    dbufs = [dbuf, dbuf2]
    idx1 = [ibufb1, ibufb1b]
    idx2 = [ibufb2, ibufb2b]
    semd = [semd0, semd1]
    seme = [seme0, seme1]

    pltpu.sync_copy(ex_hbm.at[pl.ds(ebase, SUP)], exB[0])
    pltpu.sync_copy(dstlg.at[pl.ds(ebase, SUP)], idx1[0])
    pltpu.sync_copy(srcw.at[pl.ds(ebase, SUP)], idx2[0])
    pltpu.async_copy(den_hbm.at[idx1[0]], dbufs[0], semd[0])
    pltpu.async_copy(ex_hbm.at[pl.ds(ebase + SUP, SUP)], exB[1], seme[1])
    pltpu.async_copy(dstlg.at[pl.ds(ebase + SUP, SUP)], idx1[1], seme[1])
    pltpu.async_copy(srcw.at[pl.ds(ebase + SUP, SUP)], idx2[1], seme[1])

    def super_b2(sc2, carry):
        for b in range(2):
            sc = sc2 * 2 + b
            soff = ebase + sc * SUP
            pltpu.make_async_copy(den_hbm.at[idx1[b]], dbufs[b],
                                  semd[b]).wait()

            @pl.when(sc + 1 < NSUP)
            def _ready_next():
                noff = soff + SUP
                pltpu.make_async_copy(ex_hbm.at[pl.ds(noff, SUP)],
                                      exB[1 - b], seme[1 - b]).wait()
                pltpu.make_async_copy(dstlg.at[pl.ds(noff, SUP)],
                                      idx1[1 - b], seme[1 - b]).wait()
                pltpu.make_async_copy(srcw.at[pl.ds(noff, SUP)],
                                      idx2[1 - b], seme[1 - b]).wait()
                pltpu.async_copy(den_hbm.at[idx1[1 - b]], dbufs[1 - b],
                                 semd[1 - b])

            def group_b(j, gcarry):
                base = j * LANES
                attsvec = zvec
                for ee in range(LANES):
                    alpha = (exB[b][base + ee, :] /
                             (dbufs[b][base + ee, :] + 1e-16))
                    wbuf[base + ee, :] = alpha
                    sa = jnp.sum(alpha * mask5) * (1.0 / HEADS)
                    sav = jnp.full((LANES,), sa, jnp.float32)
                    attsvec = jnp.where(iota == ee, sav, attsvec)
                attsbuf[pl.ds(base, LANES)] = attsvec
                return gcarry
            lax.fori_loop(0, SUP // LANES, group_b, 0)

            pltpu.sync_copy(wbuf, w_sp.at[idx2[b]], add=True)
            pltpu.sync_copy(attsbuf, atts_hbm.at[pl.ds(soff, SUP)])

            @pl.when(sc + 2 < NSUP)
            def _issue_next2():
                noff2 = soff + 2 * SUP
                pltpu.async_copy(ex_hbm.at[pl.ds(noff2, SUP)], exB[b],
                                 seme[b])
                pltpu.async_copy(dstlg.at[pl.ds(noff2, SUP)], idx1[b],
                                 seme[b])
                pltpu.async_copy(srcw.at[pl.ds(noff2, SUP)], idx2[b],
                                 seme[b])
        return carry
    lax.fori_loop(0, NSUP // 2, super_b2, 0)

    plsc.subcore_barrier()
    pltpu.sync_copy(w_sp.at[pl.ds(s * ZROWS, ZROWS)],
                    w_hbm.at[pl.ds(c * NSEG + s * ZROWS, ZROWS)])


def _edge_stage(xlt, xrt, srcg, dstg, dstlg, dstl, srcw, att):
    mesh = plsc.VectorSubcoreMesh(core_axis_name="c", subcore_axis_name="s",
                                  num_cores=NC, num_subcores=NS)
    out_type = [
        jax.ShapeDtypeStruct((BATCH * EP, 16), jnp.float32),
        jax.ShapeDtypeStruct((BATCH * NSEG, 16), jnp.float32),
        jax.ShapeDtypeStruct((BATCH * EP,), jnp.float32),
        jax.ShapeDtypeStruct((BATCH * NSEG, 16), jnp.float32),
    ]
    scratch = [
        pltpu.VMEM((KCH, HEADS * H), jnp.bfloat16),
        pltpu.VMEM((KCH, HEADS * H), jnp.bfloat16),
        pltpu.VMEM((KCH, HEADS * H), jnp.bfloat16),
        pltpu.VMEM((KCH, HEADS * H), jnp.bfloat16),
        pltpu.VMEM((SUP, 16), jnp.float32),
        pltpu.VMEM((SUP, 16), jnp.float32),
        pltpu.VMEM((SUP, 16), jnp.float32),
        pltpu.VMEM((SUP, 16), jnp.float32),
        pltpu.VMEM((SUP, 16), jnp.float32),
        pltpu.VMEM((SUP,), jnp.float32),
        pltpu.VMEM((SUP,), jnp.int32),
        pltpu.VMEM((SUP,), jnp.int32),
        pltpu.VMEM((SUP,), jnp.int32),
        pltpu.VMEM((SUP,), jnp.int32),
        pltpu.VMEM((SUP,), jnp.int32),
        pltpu.VMEM((SUP,), jnp.int32),
        pltpu.VMEM((SUP,), jnp.int32),
        pltpu.VMEM((HEADS, H), jnp.float32),
        pltpu.VMEM_SHARED((NSEG, 16), jnp.float32),
        pltpu.VMEM_SHARED((NSEG, 16), jnp.float32),
    ] + [pltpu.SemaphoreType.DMA] * 8
    run = pl.kernel(_sc_edge_body, out_type=out_type, mesh=mesh,
                    scratch_types=scratch,
                    compiler_params=pltpu.CompilerParams(
                        use_tc_tiling_on_sc=False,
                        needs_layout_passes=False))
    return run(xlt, xrt, srcg, dstg, dstl, dstlg, srcw, att)


def _prep_edges(edges):
    ar = jnp.arange(NN, dtype=jnp.int32)
    src = jnp.concatenate(
        [edges[:, 0, :], jnp.broadcast_to(ar, (BATCH, NN))], axis=1)
    dst = jnp.concatenate(
        [edges[:, 1, :], jnp.broadcast_to(ar, (BATCH, NN))], axis=1)
    npad = EP - (NE + NN)
    srcp = jnp.pad(src, ((0, 0), (0, npad)))
    dstp = jnp.pad(dst, ((0, 0), (0, npad)))
    pad = (jnp.arange(EP) >= (NE + NN))[None, :]
    coff = (jnp.arange(BATCH, dtype=jnp.int32) * NN)[:, None]
    srcg = (srcp + coff).astype(jnp.int32)
    dstg = (dstp + coff).astype(jnp.int32)
    dst_loc = jnp.where(pad, TRASH, dstp).astype(jnp.int32)
    dstlg = dst_loc + (jnp.arange(BATCH, dtype=jnp.int32) * NSEG)[:, None]
    srcw = jnp.where((dstp == 0) & ~pad, srcp, TRASH).astype(jnp.int32)
    return (srcg.reshape(-1), dstg.reshape(-1), dstlg.reshape(-1),
            dst_loc.reshape(-1), srcw.reshape(-1))


def _cnn(obs_map, p):
    def patches(x, kh, kw, stride):
        return lax.conv_general_dilated_patches(
            x, (kh, kw), (stride, stride), 'VALID',
            dimension_numbers=('NCHW', 'OIHW', 'NCHW'))

    def conv(x, wkey, bkey, kh, kw, stride):
        w = p[wkey]
        co = w.shape[0]
        pt = patches(x, kh, kw, stride)
        n, f, ho, wo = pt.shape
        flat = pt.transpose(0, 2, 3, 1).reshape(n * ho * wo, f)
        y = _dense(flat, w.reshape(co, f).T, p[bkey], act=True)
        return y.reshape(n, ho, wo, co).transpose(0, 3, 1, 2)

    h = conv(obs_map, 'c1W', 'c1b', 8, 8, 4)
    h = conv(h, 'c2W', 'c2b', 4, 4, 2)
    h = conv(h, 'c3W', 'c3b', 3, 3, 1)
    flat = h.reshape(h.shape[0], -1)
    return _dense(flat, p['fcW'].T, p['fcb'], act=True)


@jax.jit
def kernel(obs, obs_map, edges, edges_feature, params):
    p = params
    target_emb = _cnn(obs_map, p)
    hid = _gru_encode(obs.reshape(BATCH * NN, SEQ * OBS_IN), p)
    xlt, xrt = _project(hid, target_emb, p)

    srcg, dstg, dstlg, dstl, srcw = _prep_edges(edges)
    att4 = p['att'].reshape(HEADS, H // 32, 16, 2)
    att_sc = att4.transpose(0, 1, 3, 2).reshape(HEADS, H)
    _, _, atts_full, w_full = _edge_stage(
        xlt, xrt, srcg, dstg, dstlg, dstl, srcw, att_sc)

    atts = atts_full.reshape(BATCH, EP)[:, :NE + 1]

    w2 = (w_full.reshape(BATCH, NSEG, 16)[:, :NN, :HEADS]
          .transpose(0, 2, 1).reshape(BATCH * HEADS, 1, NN))
    o05 = _node0_agg(w2, xlt).reshape(BATCH * HEADS, H)
    q = _head(o05, p)
    return q, atts
